# baseline (speedup 1.0000x reference)
import functools

import jax
import jax.numpy as jnp
import numpy as np
from jax.experimental import pallas as pl
from jax.experimental.pallas import tpu as pltpu

_ITERS = 3
_N = 128
_K = 4
_R = 32
_P = 196
_C = 256


def _roi_kernel(f_hbm, w1t_ref, w2t_ref, woutt_ref, m_ref, xs0_ref, ys0_ref,
                out_ref, fvm, sem):
    i = pl.program_id(0)
    nsteps = pl.num_programs(0)

    def copies(step, slot):
        bsl = pl.ds(step * _R, _R)
        return [pltpu.make_async_copy(
                    f_hbm.at[p, bsl, :], fvm.at[slot, :, p, :], sem.at[slot, p])
                for p in range(_P)]

    def issue(step, slot):
        for c in copies(step, slot):
            c.start()

    def wait_all(step, slot):
        for c in copies(step, slot):
            c.wait()

    @pl.when(i == 0)
    def _():
        issue(0, 0)

    slot = i % 2
    nxt = jnp.minimum(i + 1, nsteps - 1)
    issue(nxt, 1 - slot)

    w1t = w1t_ref[...]
    w2t = w2t_ref[...]
    woutt = woutt_ref[...]
    m = m_ref[...]

    iy = jax.lax.broadcasted_iota(jnp.int32, (14, _N), 0)

    wait_all(i, slot)

    gtb = []
    for j in range(_R):
        Fb = fvm[slot, j].astype(jnp.bfloat16)
        gt = jax.lax.dot_general(w1t, Fb, (((1,), (1,)), ((), ())),
                                 preferred_element_type=jnp.float32)
        gtb.append(gt.astype(jnp.bfloat16))

    xs = [xs0_ref[...]] * _R
    ys = [ys0_ref[...]] * _R

    for _ in range(_ITERS):
        stb = []
        for r in range(_R):
            x = jnp.clip(xs[r], 0.0, 13.0 - 1e-4)
            y = jnp.clip(ys[r], 0.0, 13.0 - 1e-4)
            x0 = jnp.floor(x)
            y0 = jnp.floor(y)
            wx = x - x0
            wy = y - y0
            x0i = x0.astype(jnp.int32)
            y0i = y0.astype(jnp.int32)
            hx = (jnp.where(iy == x0i, 1.0 - wx, 0.0)
                  + jnp.where(iy == x0i + 1, wx, 0.0)).astype(jnp.bfloat16)
            hy = (jnp.where(iy == y0i, 1.0 - wy, 0.0)
                  + jnp.where(iy == y0i + 1, wy, 0.0)).astype(jnp.bfloat16)
            stb.append((hy[:, None, :] * hx[None, :, :]).reshape(_P, _N))

        ut = [jax.lax.dot(gtb[r], stb[r], preferred_element_type=jnp.float32)
              for r in range(_R)]
        mut = [jax.lax.dot(ut[r].astype(jnp.bfloat16), m,
                           preferred_element_type=jnp.float32)
               for r in range(_R)]
        ht = [jnp.maximum(mut[r], 0.0).astype(jnp.bfloat16) for r in range(_R)]

        htm = [jax.lax.dot(ht[r], m, preferred_element_type=jnp.float32)
               for r in range(_R)]
        h2t = [jnp.maximum(
                   jax.lax.dot(w2t, htm[r].astype(jnp.bfloat16),
                               preferred_element_type=jnp.float32),
                   0.0).astype(jnp.bfloat16)
               for r in range(_R)]

        offt = [jnp.tanh(jax.lax.dot(woutt, h2t[r],
                                     preferred_element_type=jnp.float32))
                for r in range(_R)]
        for r in range(_R):
            xs[r] = xs[r] + offt[r][0:1, :]
            ys[r] = ys[r] + offt[r][1:2, :]

    for r in range(_R):
        out_ref[r] = jnp.concatenate([xs[r], ys[r]], axis=0)

    @pl.when(i == nsteps - 1)
    def _():
        wait_all(nxt, 1 - (i % 2))


@functools.partial(jax.jit, static_argnames=())
def kernel(features, W1, W2, Wout):
    B, C, H, W = features.shape
    n = _N
    feats = jnp.transpose(features, (2, 3, 0, 1)).reshape(H * W, B, C)

    angles = 2.0 * np.pi * np.arange(n, dtype=np.float32) / n
    cx = (W - 1) / 2.0
    cy = (H - 1) / 2.0
    xs0 = jnp.asarray((cx + 0.4 * (W - 1) * np.cos(angles))[None, :],
                      dtype=jnp.float32)
    ys0 = jnp.asarray((cy + 0.4 * (H - 1) * np.sin(angles))[None, :],
                      dtype=jnp.float32)

    mat = np.eye(n, dtype=np.float32)
    idx = np.arange(n)
    for o in (-2, -1, 1, 2):
        mat[idx, (idx + o) % n] += 1.0 / _K
    m_const = jnp.asarray(mat, dtype=jnp.bfloat16)

    w1t = W1.T.astype(jnp.bfloat16)
    w2t = W2.T.astype(jnp.bfloat16)
    woutt = jnp.zeros((8, W2.shape[1]), jnp.float32).at[0:2].set(Wout.T)
    woutt = woutt.astype(jnp.bfloat16)

    out = pl.pallas_call(
        _roi_kernel,
        grid=(B // _R,),
        in_specs=[
            pl.BlockSpec(memory_space=pl.ANY),
            pl.BlockSpec((n, C), lambda i: (0, 0)),
            pl.BlockSpec((n, n), lambda i: (0, 0)),
            pl.BlockSpec((8, n), lambda i: (0, 0)),
            pl.BlockSpec((n, n), lambda i: (0, 0)),
            pl.BlockSpec((1, n), lambda i: (0, 0)),
            pl.BlockSpec((1, n), lambda i: (0, 0)),
        ],
        out_specs=pl.BlockSpec((_R, 2, n), lambda i: (i, 0, 0)),
        out_shape=jax.ShapeDtypeStruct((B, 2, n), jnp.float32),
        scratch_shapes=[
            pltpu.VMEM((2, _R, _P, _C), jnp.float32),
            pltpu.SemaphoreType.DMA((2, _P)),
        ],
        compiler_params=pltpu.CompilerParams(
            dimension_semantics=("arbitrary",),
        ),
    )(feats, w1t, w2t, woutt, m_const, xs0, ys0)

    return jnp.swapaxes(out, 1, 2)


# speedup vs baseline: 1.4219x; 1.4219x over previous
import functools

import jax
import jax.numpy as jnp
import numpy as np
from jax.experimental import pallas as pl
from jax.experimental.pallas import tpu as pltpu

_ITERS = 3
_N = 128
_K = 4
_R = 32
_P = 196
_PP = 224
_C = 256


def _roi_kernel(f_hbm, w1t_ref, w2t_ref, woutt_ref, m_ref, xs0_ref, ys0_ref,
                out_ref, fvm, sem):
    i = pl.program_id(0)
    nsteps = pl.num_programs(0)

    def copies(step, slot):
        bsl = pl.ds(step * _R, _R)
        return [pltpu.make_async_copy(
                    f_hbm.at[p, bsl, :],
                    fvm.at[slot, :, (p // 14) * 16 + p % 14, :],
                    sem.at[slot, p])
                for p in range(_P)]

    def issue(step, slot):
        for c in copies(step, slot):
            c.start()

    def wait_all(step, slot):
        for c in copies(step, slot):
            c.wait()

    @pl.when(i == 0)
    def _():
        zpad = jnp.zeros((_R, 2, _C), jnp.float32)
        for s_ in range(2):
            for k in range(14):
                fvm[s_, :, 16 * k + 14:16 * k + 16, :] = zpad
        issue(0, 0)

    @pl.when(i + 1 < nsteps)
    def _():
        issue(i + 1, (i + 1) % 2)

    slot = i % 2

    w1t = w1t_ref[...]
    w2t = w2t_ref[...]
    woutt = woutt_ref[...]
    m = m_ref[...]

    iy = jax.lax.broadcasted_iota(jnp.int32, (14, _N), 0)
    ix = jax.lax.broadcasted_iota(jnp.int32, (16, _N), 0)

    wait_all(i, slot)

    gtb = []
    for j in range(_R):
        Fb = fvm[slot, j].astype(jnp.bfloat16)
        gt = jax.lax.dot_general(w1t, Fb, (((1,), (1,)), ((), ())),
                                 preferred_element_type=jnp.float32)
        gtb.append(gt.astype(jnp.bfloat16))

    xs = [xs0_ref[...]] * _R
    ys = [ys0_ref[...]] * _R

    for _ in range(_ITERS):
        stb = []
        for r in range(_R):
            x = jnp.clip(xs[r], 0.0, 13.0 - 1e-4)
            y = jnp.clip(ys[r], 0.0, 13.0 - 1e-4)
            x0 = jnp.floor(x)
            y0 = jnp.floor(y)
            wx = x - x0
            wy = y - y0
            x0i = x0.astype(jnp.int32)
            y0i = y0.astype(jnp.int32)
            hx = (jnp.where(ix == x0i, 1.0 - wx, 0.0)
                  + jnp.where(ix == x0i + 1, wx, 0.0)).astype(jnp.bfloat16)
            hy = (jnp.where(iy == y0i, 1.0 - wy, 0.0)
                  + jnp.where(iy == y0i + 1, wy, 0.0)).astype(jnp.bfloat16)
            stb.append((hy[:, None, :] * hx[None, :, :]).reshape(_PP, _N))

        ut = [jax.lax.dot(gtb[r], stb[r], preferred_element_type=jnp.float32)
              for r in range(_R)]
        mut = [jax.lax.dot(ut[r].astype(jnp.bfloat16), m,
                           preferred_element_type=jnp.float32)
               for r in range(_R)]
        ht = [jnp.maximum(mut[r], 0.0).astype(jnp.bfloat16) for r in range(_R)]

        htm = [jax.lax.dot(ht[r], m, preferred_element_type=jnp.float32)
               for r in range(_R)]
        h2t = [jnp.maximum(
                   jax.lax.dot(w2t, htm[r].astype(jnp.bfloat16),
                               preferred_element_type=jnp.float32),
                   0.0).astype(jnp.bfloat16)
               for r in range(_R)]

        offt = [jnp.tanh(jax.lax.dot(woutt, h2t[r],
                                     preferred_element_type=jnp.float32))
                for r in range(_R)]
        for r in range(_R):
            xs[r] = xs[r] + offt[r][0:1, :]
            ys[r] = ys[r] + offt[r][1:2, :]

    for r in range(_R):
        out_ref[r] = jnp.concatenate([xs[r], ys[r]], axis=0)


@functools.partial(jax.jit, static_argnames=())
def kernel(features, W1, W2, Wout):
    B, C, H, W = features.shape
    n = _N
    feats = jnp.transpose(features, (2, 3, 0, 1)).reshape(H * W, B, C)

    angles = 2.0 * np.pi * np.arange(n, dtype=np.float32) / n
    cx = (W - 1) / 2.0
    cy = (H - 1) / 2.0
    xs0 = jnp.asarray((cx + 0.4 * (W - 1) * np.cos(angles))[None, :],
                      dtype=jnp.float32)
    ys0 = jnp.asarray((cy + 0.4 * (H - 1) * np.sin(angles))[None, :],
                      dtype=jnp.float32)

    mat = np.eye(n, dtype=np.float32)
    idx = np.arange(n)
    for o in (-2, -1, 1, 2):
        mat[idx, (idx + o) % n] += 1.0 / _K
    m_const = jnp.asarray(mat, dtype=jnp.bfloat16)

    w1t = W1.T.astype(jnp.bfloat16)
    w2t = W2.T.astype(jnp.bfloat16)
    woutt = jnp.zeros((8, W2.shape[1]), jnp.float32).at[0:2].set(Wout.T)
    woutt = woutt.astype(jnp.bfloat16)

    out = pl.pallas_call(
        _roi_kernel,
        grid=(B // _R,),
        in_specs=[
            pl.BlockSpec(memory_space=pl.ANY),
            pl.BlockSpec((n, C), lambda i: (0, 0)),
            pl.BlockSpec((n, n), lambda i: (0, 0)),
            pl.BlockSpec((8, n), lambda i: (0, 0)),
            pl.BlockSpec((n, n), lambda i: (0, 0)),
            pl.BlockSpec((1, n), lambda i: (0, 0)),
            pl.BlockSpec((1, n), lambda i: (0, 0)),
        ],
        out_specs=pl.BlockSpec((_R, 2, n), lambda i: (i, 0, 0)),
        out_shape=jax.ShapeDtypeStruct((B, 2, n), jnp.float32),
        scratch_shapes=[
            pltpu.VMEM((2, _R, _PP, _C), jnp.float32),
            pltpu.SemaphoreType.DMA((2, _P)),
        ],
        compiler_params=pltpu.CompilerParams(
            dimension_semantics=("arbitrary",),
        ),
    )(feats, w1t, w2t, woutt, m_const, xs0, ys0)

    return jnp.swapaxes(out, 1, 2)


# speedup vs baseline: 1.5862x; 1.1155x over previous
import functools

import jax
import jax.numpy as jnp
import numpy as np
from jax.experimental import pallas as pl
from jax.experimental.pallas import tpu as pltpu

_ITERS = 3
_N = 128
_K = 4
_R = 64
_P = 196
_PP = 224
_C = 256


def _roi_kernel(f_hbm, w1t_ref, w2t_ref, woutt_ref, m_ref, xs0_ref, ys0_ref,
                out_ref, fvm, sem):
    i = pl.program_id(0)
    nsteps = pl.num_programs(0)

    def copies(step, slot):
        bsl = pl.ds(step * _R, _R)
        return [pltpu.make_async_copy(
                    f_hbm.at[p, bsl, :],
                    fvm.at[slot, :, (p // 14) * 16 + p % 14, :],
                    sem.at[slot, p])
                for p in range(_P)]

    def issue(step, slot):
        for c in copies(step, slot):
            c.start()

    def wait_all(step, slot):
        for c in copies(step, slot):
            c.wait()

    @pl.when(i == 0)
    def _():
        zpad = jnp.zeros((_R, 2, _C), jnp.float32)
        for s_ in range(2):
            for k in range(14):
                fvm[s_, :, 16 * k + 14:16 * k + 16, :] = zpad
        issue(0, 0)

    par = i % 2
    @pl.when(jnp.logical_and(i + 1 < nsteps, par == 0))
    def _():
        issue(i + 1, 1)

    @pl.when(jnp.logical_and(i + 1 < nsteps, par == 1))
    def _():
        issue(i + 1, 0)

    w1t = w1t_ref[...]
    w2t = w2t_ref[...]
    woutt = woutt_ref[...]
    m = m_ref[...]

    iy = jax.lax.broadcasted_iota(jnp.int32, (14, _N), 0)
    ix = jax.lax.broadcasted_iota(jnp.int32, (16, _N), 0)

    @pl.when(par == 0)
    def _():
        wait_all(i, 0)

    @pl.when(par == 1)
    def _():
        wait_all(i, 1)

    slot = par
    gtb = []
    for j in range(_R):
        Fb = fvm[slot, j].astype(jnp.bfloat16)
        gt = jax.lax.dot_general(w1t, Fb, (((1,), (1,)), ((), ())),
                                 preferred_element_type=jnp.float32)
        gtb.append(gt.astype(jnp.bfloat16))

    xs = [xs0_ref[...]] * _R
    ys = [ys0_ref[...]] * _R

    for _ in range(_ITERS):
        stb = []
        for r in range(_R):
            x = jnp.clip(xs[r], 0.0, 13.0 - 1e-4)
            y = jnp.clip(ys[r], 0.0, 13.0 - 1e-4)
            x0 = jnp.floor(x)
            y0 = jnp.floor(y)
            wx = x - x0
            wy = y - y0
            x0i = x0.astype(jnp.int32)
            y0i = y0.astype(jnp.int32)
            hx = (jnp.where(ix == x0i, 1.0 - wx, 0.0)
                  + jnp.where(ix == x0i + 1, wx, 0.0)).astype(jnp.bfloat16)
            hy = (jnp.where(iy == y0i, 1.0 - wy, 0.0)
                  + jnp.where(iy == y0i + 1, wy, 0.0)).astype(jnp.bfloat16)
            stb.append((hy[:, None, :] * hx[None, :, :]).reshape(_PP, _N))

        ut = [jax.lax.dot(gtb[r], stb[r], preferred_element_type=jnp.float32)
              for r in range(_R)]
        mut = [jax.lax.dot(ut[r].astype(jnp.bfloat16), m,
                           preferred_element_type=jnp.float32)
               for r in range(_R)]
        ht = [jnp.maximum(mut[r], 0.0).astype(jnp.bfloat16) for r in range(_R)]

        htm = [jax.lax.dot(ht[r], m, preferred_element_type=jnp.float32)
               for r in range(_R)]
        h2t = [jnp.maximum(
                   jax.lax.dot(w2t, htm[r].astype(jnp.bfloat16),
                               preferred_element_type=jnp.float32),
                   0.0).astype(jnp.bfloat16)
               for r in range(_R)]

        offt = [jnp.tanh(jax.lax.dot(woutt, h2t[r],
                                     preferred_element_type=jnp.float32))
                for r in range(_R)]
        for r in range(_R):
            xs[r] = xs[r] + offt[r][0:1, :]
            ys[r] = ys[r] + offt[r][1:2, :]

    for r in range(_R):
        out_ref[r] = jnp.concatenate([xs[r], ys[r]], axis=0)


@functools.partial(jax.jit, static_argnames=())
def kernel(features, W1, W2, Wout):
    B, C, H, W = features.shape
    n = _N
    feats = jnp.transpose(features, (2, 3, 0, 1)).reshape(H * W, B, C)

    angles = 2.0 * np.pi * np.arange(n, dtype=np.float32) / n
    cx = (W - 1) / 2.0
    cy = (H - 1) / 2.0
    xs0 = jnp.asarray((cx + 0.4 * (W - 1) * np.cos(angles))[None, :],
                      dtype=jnp.float32)
    ys0 = jnp.asarray((cy + 0.4 * (H - 1) * np.sin(angles))[None, :],
                      dtype=jnp.float32)

    mat = np.eye(n, dtype=np.float32)
    idx = np.arange(n)
    for o in (-2, -1, 1, 2):
        mat[idx, (idx + o) % n] += 1.0 / _K
    m_const = jnp.asarray(mat, dtype=jnp.bfloat16)

    w1t = W1.T.astype(jnp.bfloat16)
    w2t = W2.T.astype(jnp.bfloat16)
    woutt = jnp.zeros((8, W2.shape[1]), jnp.float32).at[0:2].set(Wout.T)
    woutt = woutt.astype(jnp.bfloat16)

    out = pl.pallas_call(
        _roi_kernel,
        grid=(B // _R,),
        in_specs=[
            pl.BlockSpec(memory_space=pl.ANY),
            pl.BlockSpec((n, C), lambda i: (0, 0)),
            pl.BlockSpec((n, n), lambda i: (0, 0)),
            pl.BlockSpec((8, n), lambda i: (0, 0)),
            pl.BlockSpec((n, n), lambda i: (0, 0)),
            pl.BlockSpec((1, n), lambda i: (0, 0)),
            pl.BlockSpec((1, n), lambda i: (0, 0)),
        ],
        out_specs=pl.BlockSpec((_R, 2, n), lambda i: (i, 0, 0)),
        out_shape=jax.ShapeDtypeStruct((B, 2, n), jnp.float32),
        scratch_shapes=[
            pltpu.VMEM((2, _R, _PP, _C), jnp.float32),
            pltpu.SemaphoreType.DMA((2, _P)),
        ],
        compiler_params=pltpu.CompilerParams(
            dimension_semantics=("arbitrary",),
        ),
    )(feats, w1t, w2t, woutt, m_const, xs0, ys0)

    return jnp.swapaxes(out, 1, 2)
